# baseline (speedup 1.0000x reference)
import jax
import jax.numpy as jnp
from jax import lax
from jax.experimental import pallas as pl
from jax.experimental.pallas import tpu as pltpu
from jax.experimental.pallas import tpu_sc as plsc

_NW = 32
_CH = 128


def _seg_sum(g, src_r, dst_r, zrows, n_pad):
    n, d = g.shape
    cpw = src_r.shape[1]
    rpt = n_pad // 16

    def body(g_hbm, src_hbm, dst_hbm, z_hbm, out_hbm, src_v, dst_v, rows_v,
             acc, sem):
        c = lax.axis_index("c")
        s = lax.axis_index("s")
        w = s * 2 + c
        pltpu.sync_copy(src_hbm.at[w], src_v)
        pltpu.sync_copy(dst_hbm.at[w], dst_v)
        pltpu.sync_copy(z_hbm, acc.at[pl.ds(s * rpt, rpt)])
        plsc.subcore_barrier()

        def step(j, carry):
            pltpu.async_copy(g_hbm.at[src_v.at[j]], rows_v, sem).wait()
            pltpu.sync_copy(rows_v, acc.at[dst_v.at[j]], add=True)
            return carry

        lax.fori_loop(0, cpw, step, 0)
        plsc.subcore_barrier()
        pltpu.sync_copy(acc.at[pl.ds(s * rpt, rpt)],
                        out_hbm.at[pl.ds(c * n_pad + s * rpt, rpt)])

    return pl.kernel(
        body,
        out_type=jax.ShapeDtypeStruct((2 * n_pad, d), jnp.float32),
        mesh=plsc.VectorSubcoreMesh(core_axis_name="c", subcore_axis_name="s"),
        scratch_types=[
            pltpu.VMEM((cpw, _CH), jnp.int32),
            pltpu.VMEM((cpw, _CH), jnp.int32),
            pltpu.VMEM((_CH, d), jnp.float32),
            pltpu.VMEM_SHARED((n_pad, d), jnp.float32),
            pltpu.SemaphoreType.DMA,
        ],
    )(g, src_r, dst_r, zrows)


_R = 1000


def _dis_block(degp_ref):
    deg = degp_ref[0, :, :1] + degp_ref[1, :, :1]
    return jnp.where(deg > 0, lax.rsqrt(jnp.maximum(deg, 1.0)), 0.0)


def _tc_pre(x, w, b, degp, n_pad):
    n, d = x.shape

    def body(x_ref, w_ref, b_ref, degp_ref, h_ref, g_ref):
        h = jnp.dot(x_ref[...], w_ref[...], preferred_element_type=jnp.float32)
        h = jnp.maximum(h + b_ref[...], 0.0)
        dis = _dis_block(degp_ref)
        h_ref[...] = h
        g_ref[...] = h * dis

    return pl.pallas_call(
        body,
        grid=(n // _R,),
        in_specs=[
            pl.BlockSpec((_R, d), lambda i: (i, 0)),
            pl.BlockSpec((d, d), lambda i: (0, 0)),
            pl.BlockSpec((1, d), lambda i: (0, 0)),
            pl.BlockSpec((2, _R, 16), lambda i: (0, i, 0)),
        ],
        out_specs=[pl.BlockSpec((_R, d), lambda i: (i, 0))] * 2,
        out_shape=[jax.ShapeDtypeStruct((n, d), jnp.float32)] * 2,
    )(x, w, b, degp.reshape(2, n_pad, -1))


def _tc_combine(p, degp, n, n_pad):
    d = p.shape[-1]

    def body(p_ref, degp_ref, g_ref):
        dis = _dis_block(degp_ref)
        g_ref[...] = (p_ref[0] + p_ref[1]) * (dis * dis)

    return pl.pallas_call(
        body,
        grid=(n // _R,),
        in_specs=[
            pl.BlockSpec((2, _R, d), lambda i: (0, i, 0)),
            pl.BlockSpec((2, _R, 16), lambda i: (0, i, 0)),
        ],
        out_specs=pl.BlockSpec((_R, d), lambda i: (i, 0)),
        out_shape=jax.ShapeDtypeStruct((n, d), jnp.float32),
    )(p.reshape(2, n_pad, d), degp.reshape(2, n_pad, -1))


def _tag_block(h_ref, p_refs, degp_ref, wk_ref, b_ref, gam_ref, bet_ref):
    dis = _dis_block(degp_ref)
    acc = jnp.dot(h_ref[...], wk_ref[0], preferred_element_type=jnp.float32)
    for k, p_ref in enumerate(p_refs, start=1):
        hk = (p_ref[0] + p_ref[1]) * dis
        acc = acc + jnp.dot(hk, wk_ref[k], preferred_element_type=jnp.float32)
    acc = jnp.maximum(acc + b_ref[...], 0.0)
    mu = jnp.mean(acc, axis=-1, keepdims=True)
    var = jnp.mean((acc - mu) ** 2, axis=-1, keepdims=True)
    y = (acc - mu) / jnp.sqrt(var + 1e-5) * gam_ref[...] + bet_ref[...]
    return y, dis


def _tc_tag_final(h, p1, p2, p3, degp, wk, b, gam, bet, n_pad):
    n, d = h.shape
    kk = wk.shape[0]

    def body(h_ref, p1_ref, p2_ref, p3_ref, degp_ref, wk_ref, b_ref, gam_ref,
             bet_ref, hn_ref, gn_ref):
        y, dis = _tag_block(h_ref, (p1_ref, p2_ref, p3_ref), degp_ref, wk_ref,
                            b_ref, gam_ref, bet_ref)
        hn_ref[...] = y
        gn_ref[...] = y * dis

    return pl.pallas_call(
        body,
        grid=(n // _R,),
        in_specs=[
            pl.BlockSpec((_R, d), lambda i: (i, 0)),
            pl.BlockSpec((2, _R, d), lambda i: (0, i, 0)),
            pl.BlockSpec((2, _R, d), lambda i: (0, i, 0)),
            pl.BlockSpec((2, _R, d), lambda i: (0, i, 0)),
            pl.BlockSpec((2, _R, 16), lambda i: (0, i, 0)),
            pl.BlockSpec((kk, d, d), lambda i: (0, 0, 0)),
            pl.BlockSpec((1, d), lambda i: (0, 0)),
            pl.BlockSpec((1, d), lambda i: (0, 0)),
            pl.BlockSpec((1, d), lambda i: (0, 0)),
        ],
        out_specs=[pl.BlockSpec((_R, d), lambda i: (i, 0))] * 2,
        out_shape=[jax.ShapeDtypeStruct((n, d), jnp.float32)] * 2,
    )(h, p1.reshape(2, n_pad, d), p2.reshape(2, n_pad, d),
      p3.reshape(2, n_pad, d), degp.reshape(2, n_pad, -1), wk,
      b.reshape(1, d), gam.reshape(1, d), bet.reshape(1, d))


def _tc_tag_final_post(h, p1, p2, p3, degp, wk, b, gam, bet,
                       w_po1, b_po1, w_po2, b_po2, n_pad):
    n, d = h.shape
    kk = wk.shape[0]

    def body(h_ref, p1_ref, p2_ref, p3_ref, degp_ref, wk_ref, b_ref, gam_ref,
             bet_ref, w1_ref, b1_ref, w2_ref, b2_ref, out_ref):
        y, _ = _tag_block(h_ref, (p1_ref, p2_ref, p3_ref), degp_ref, wk_ref,
                          b_ref, gam_ref, bet_ref)
        hp = jnp.dot(y, w1_ref[...], preferred_element_type=jnp.float32)
        hp = jnp.maximum(hp + b1_ref[...], 0.0)
        out_ref[...] = (jnp.dot(hp, w2_ref[...],
                                preferred_element_type=jnp.float32)
                        + b2_ref[...])

    return pl.pallas_call(
        body,
        grid=(n // _R,),
        in_specs=[
            pl.BlockSpec((_R, d), lambda i: (i, 0)),
            pl.BlockSpec((2, _R, d), lambda i: (0, i, 0)),
            pl.BlockSpec((2, _R, d), lambda i: (0, i, 0)),
            pl.BlockSpec((2, _R, d), lambda i: (0, i, 0)),
            pl.BlockSpec((2, _R, 16), lambda i: (0, i, 0)),
            pl.BlockSpec((kk, d, d), lambda i: (0, 0, 0)),
            pl.BlockSpec((1, d), lambda i: (0, 0)),
            pl.BlockSpec((1, d), lambda i: (0, 0)),
            pl.BlockSpec((1, d), lambda i: (0, 0)),
            pl.BlockSpec((d, d), lambda i: (0, 0)),
            pl.BlockSpec((1, d), lambda i: (0, 0)),
            pl.BlockSpec((d, 1), lambda i: (0, 0)),
            pl.BlockSpec((1, 1), lambda i: (0, 0)),
        ],
        out_specs=pl.BlockSpec((_R, 1), lambda i: (i, 0)),
        out_shape=jax.ShapeDtypeStruct((n, 1), jnp.float32),
    )(h, p1.reshape(2, n_pad, d), p2.reshape(2, n_pad, d),
      p3.reshape(2, n_pad, d), degp.reshape(2, n_pad, -1), wk,
      b.reshape(1, d), gam.reshape(1, d), bet.reshape(1, d),
      w_po1, b_po1.reshape(1, d), w_po2, b_po2.reshape(1, 1))


def kernel(x, edge_index, W_pre, b_pre, W_mp1, b_mp1, g1, bt1,
           W_mp2, b_mp2, g2, bt2, W_po1, b_po1, W_po2, b_po2):
    n, d = x.shape
    e = edge_index.shape[1]

    cpw = -(-e // (_NW * _CH))
    e_pad = _NW * _CH * cpw
    n_pad = -(-(n + 1) // 128) * 128

    pad_dst = n + (jnp.arange(e_pad - e, dtype=jnp.int32) % (n_pad - n))
    src_r = jnp.pad(edge_index[0], (0, e_pad - e)).reshape(_NW, cpw, _CH)
    dst_r = jnp.concatenate([edge_index[1], pad_dst]).reshape(_NW, cpw, _CH)

    rpt = n_pad // 16
    zrows = jnp.zeros((rpt, d), jnp.float32)
    ones_n = jnp.ones((n, d), jnp.float32)

    degp = _seg_sum(ones_n, src_r, dst_r, zrows, n_pad)
    degp = degp.reshape(2, n_pad, d)[:, :, :16]

    h, g = _tc_pre(x, W_pre, b_pre.reshape(1, d), degp, n_pad)

    for wk, bb, gam, bet, last in ((W_mp1, b_mp1, g1, bt1, False),
                                   (W_mp2, b_mp2, g2, bt2, True)):
        p1 = _seg_sum(g, src_r, dst_r, zrows, n_pad)
        g = _tc_combine(p1, degp, n, n_pad)
        p2 = _seg_sum(g, src_r, dst_r, zrows, n_pad)
        g = _tc_combine(p2, degp, n, n_pad)
        p3 = _seg_sum(g, src_r, dst_r, zrows, n_pad)
        if last:
            return _tc_tag_final_post(h, p1, p2, p3, degp, wk, bb, gam, bet,
                                      W_po1, b_po1, W_po2, b_po2, n_pad)
        h, g = _tc_tag_final(h, p1, p2, p3, degp, wk, bb, gam, bet, n_pad)


# speedup vs baseline: 1.0701x; 1.0701x over previous
import jax
import jax.numpy as jnp
from jax import lax
from jax.experimental import pallas as pl
from jax.experimental.pallas import tpu as pltpu
from jax.experimental.pallas import tpu_sc as plsc

_NW = 32
_CH = 128


def _seg_sum(g, src_r, dst_r, zrows, n_pad, cpw0, cpw1):
    n, d = g.shape
    rpt = n_pad // 16

    def body(g_hbm, src_hbm, dst_hbm, z_hbm, out_hbm, src_v, dst_v, rows_v,
             acc, sem):
        c = lax.axis_index("c")
        s = lax.axis_index("s")
        pltpu.sync_copy(src_hbm.at[c, s], src_v)
        pltpu.sync_copy(dst_hbm.at[c, s], dst_v)
        pltpu.sync_copy(z_hbm, acc.at[pl.ds(s * rpt, rpt)])
        plsc.subcore_barrier()

        def step(j, carry):
            pltpu.async_copy(g_hbm.at[src_v.at[j]], rows_v, sem).wait()
            pltpu.sync_copy(rows_v, acc.at[dst_v.at[j]], add=True)
            return carry

        lax.fori_loop(0, jnp.where(c == 0, cpw0, cpw1), step, 0)
        plsc.subcore_barrier()
        pltpu.sync_copy(acc.at[pl.ds(s * rpt, rpt)],
                        out_hbm.at[pl.ds(c * n_pad + s * rpt, rpt)])

    return pl.kernel(
        body,
        out_type=jax.ShapeDtypeStruct((2 * n_pad, d), jnp.float32),
        mesh=plsc.VectorSubcoreMesh(core_axis_name="c", subcore_axis_name="s"),
        scratch_types=[
            pltpu.VMEM((src_r.shape[2], _CH), jnp.int32),
            pltpu.VMEM((src_r.shape[2], _CH), jnp.int32),
            pltpu.VMEM((_CH, d), jnp.float32),
            pltpu.VMEM_SHARED((n_pad, d), jnp.float32),
            pltpu.SemaphoreType.DMA,
        ],
    )(g, src_r, dst_r, zrows)


_R = 1000


def _dis_block(degp_ref):
    deg = degp_ref[0, :, :1] + degp_ref[1, :, :1]
    return jnp.where(deg > 0, lax.rsqrt(jnp.maximum(deg, 1.0)), 0.0)


def _tc_pre(x, w, b, degp, n_pad):
    n, d = x.shape

    def body(x_ref, w_ref, b_ref, degp_ref, h_ref, g_ref):
        h = jnp.dot(x_ref[...], w_ref[...], preferred_element_type=jnp.float32)
        h = jnp.maximum(h + b_ref[...], 0.0)
        dis = _dis_block(degp_ref)
        h_ref[...] = h
        g_ref[...] = h * dis

    return pl.pallas_call(
        body,
        grid=(n // _R,),
        in_specs=[
            pl.BlockSpec((_R, d), lambda i: (i, 0)),
            pl.BlockSpec((d, d), lambda i: (0, 0)),
            pl.BlockSpec((1, d), lambda i: (0, 0)),
            pl.BlockSpec((2, _R, 16), lambda i: (0, i, 0)),
        ],
        out_specs=[pl.BlockSpec((_R, d), lambda i: (i, 0))] * 2,
        out_shape=[jax.ShapeDtypeStruct((n, d), jnp.float32)] * 2,
    )(x, w, b, degp.reshape(2, n_pad, -1))


def _tc_combine(p, degp, n, n_pad):
    d = p.shape[-1]

    def body(p_ref, degp_ref, g_ref):
        dis = _dis_block(degp_ref)
        g_ref[...] = (p_ref[0] + p_ref[1]) * (dis * dis)

    return pl.pallas_call(
        body,
        grid=(n // _R,),
        in_specs=[
            pl.BlockSpec((2, _R, d), lambda i: (0, i, 0)),
            pl.BlockSpec((2, _R, 16), lambda i: (0, i, 0)),
        ],
        out_specs=pl.BlockSpec((_R, d), lambda i: (i, 0)),
        out_shape=jax.ShapeDtypeStruct((n, d), jnp.float32),
    )(p.reshape(2, n_pad, d), degp.reshape(2, n_pad, -1))


def _tag_block(h_ref, p_refs, degp_ref, wk_ref, b_ref, gam_ref, bet_ref):
    dis = _dis_block(degp_ref)
    acc = jnp.dot(h_ref[...], wk_ref[0], preferred_element_type=jnp.float32)
    for k, p_ref in enumerate(p_refs, start=1):
        hk = (p_ref[0] + p_ref[1]) * dis
        acc = acc + jnp.dot(hk, wk_ref[k], preferred_element_type=jnp.float32)
    acc = jnp.maximum(acc + b_ref[...], 0.0)
    mu = jnp.mean(acc, axis=-1, keepdims=True)
    var = jnp.mean((acc - mu) ** 2, axis=-1, keepdims=True)
    y = (acc - mu) / jnp.sqrt(var + 1e-5) * gam_ref[...] + bet_ref[...]
    return y, dis


def _tc_tag_final(h, p1, p2, p3, degp, wk, b, gam, bet, n_pad):
    n, d = h.shape
    kk = wk.shape[0]

    def body(h_ref, p1_ref, p2_ref, p3_ref, degp_ref, wk_ref, b_ref, gam_ref,
             bet_ref, hn_ref, gn_ref):
        y, dis = _tag_block(h_ref, (p1_ref, p2_ref, p3_ref), degp_ref, wk_ref,
                            b_ref, gam_ref, bet_ref)
        hn_ref[...] = y
        gn_ref[...] = y * dis

    return pl.pallas_call(
        body,
        grid=(n // _R,),
        in_specs=[
            pl.BlockSpec((_R, d), lambda i: (i, 0)),
            pl.BlockSpec((2, _R, d), lambda i: (0, i, 0)),
            pl.BlockSpec((2, _R, d), lambda i: (0, i, 0)),
            pl.BlockSpec((2, _R, d), lambda i: (0, i, 0)),
            pl.BlockSpec((2, _R, 16), lambda i: (0, i, 0)),
            pl.BlockSpec((kk, d, d), lambda i: (0, 0, 0)),
            pl.BlockSpec((1, d), lambda i: (0, 0)),
            pl.BlockSpec((1, d), lambda i: (0, 0)),
            pl.BlockSpec((1, d), lambda i: (0, 0)),
        ],
        out_specs=[pl.BlockSpec((_R, d), lambda i: (i, 0))] * 2,
        out_shape=[jax.ShapeDtypeStruct((n, d), jnp.float32)] * 2,
    )(h, p1.reshape(2, n_pad, d), p2.reshape(2, n_pad, d),
      p3.reshape(2, n_pad, d), degp.reshape(2, n_pad, -1), wk,
      b.reshape(1, d), gam.reshape(1, d), bet.reshape(1, d))


def _tc_tag_final_post(h, p1, p2, p3, degp, wk, b, gam, bet,
                       w_po1, b_po1, w_po2, b_po2, n_pad):
    n, d = h.shape
    kk = wk.shape[0]

    def body(h_ref, p1_ref, p2_ref, p3_ref, degp_ref, wk_ref, b_ref, gam_ref,
             bet_ref, w1_ref, b1_ref, w2_ref, b2_ref, out_ref):
        y, _ = _tag_block(h_ref, (p1_ref, p2_ref, p3_ref), degp_ref, wk_ref,
                          b_ref, gam_ref, bet_ref)
        hp = jnp.dot(y, w1_ref[...], preferred_element_type=jnp.float32)
        hp = jnp.maximum(hp + b1_ref[...], 0.0)
        out_ref[...] = (jnp.dot(hp, w2_ref[...],
                                preferred_element_type=jnp.float32)
                        + b2_ref[...])

    return pl.pallas_call(
        body,
        grid=(n // _R,),
        in_specs=[
            pl.BlockSpec((_R, d), lambda i: (i, 0)),
            pl.BlockSpec((2, _R, d), lambda i: (0, i, 0)),
            pl.BlockSpec((2, _R, d), lambda i: (0, i, 0)),
            pl.BlockSpec((2, _R, d), lambda i: (0, i, 0)),
            pl.BlockSpec((2, _R, 16), lambda i: (0, i, 0)),
            pl.BlockSpec((kk, d, d), lambda i: (0, 0, 0)),
            pl.BlockSpec((1, d), lambda i: (0, 0)),
            pl.BlockSpec((1, d), lambda i: (0, 0)),
            pl.BlockSpec((1, d), lambda i: (0, 0)),
            pl.BlockSpec((d, d), lambda i: (0, 0)),
            pl.BlockSpec((1, d), lambda i: (0, 0)),
            pl.BlockSpec((d, 1), lambda i: (0, 0)),
            pl.BlockSpec((1, 1), lambda i: (0, 0)),
        ],
        out_specs=pl.BlockSpec((_R, 1), lambda i: (i, 0)),
        out_shape=jax.ShapeDtypeStruct((n, 1), jnp.float32),
    )(h, p1.reshape(2, n_pad, d), p2.reshape(2, n_pad, d),
      p3.reshape(2, n_pad, d), degp.reshape(2, n_pad, -1), wk,
      b.reshape(1, d), gam.reshape(1, d), bet.reshape(1, d),
      w_po1, b_po1.reshape(1, d), w_po2, b_po2.reshape(1, 1))


def kernel(x, edge_index, W_pre, b_pre, W_mp1, b_mp1, g1, bt1,
           W_mp2, b_mp2, g2, bt2, W_po1, b_po1, W_po2, b_po2):
    n, d = x.shape
    e = edge_index.shape[1]

    n_pad = -(-(n + 1) // 128) * 128

    ept = 16 * _CH
    cpw0 = max(1, round(e * 0.37 / ept))
    e0 = ept * cpw0
    cpw1 = -(-(e - e0) // ept)
    cpw_max = max(cpw0, cpw1)

    e_pad1 = ept * cpw1 - (e - e0)
    pad_dst = n + (jnp.arange(e_pad1, dtype=jnp.int32) % (n_pad - n))

    def _core_split(idx, pad_vals):
        a0 = idx[:e0].reshape(16, cpw0, _CH)
        a0 = jnp.pad(a0, ((0, 0), (0, cpw_max - cpw0), (0, 0)))
        a1 = jnp.concatenate([idx[e0:], pad_vals]).reshape(16, cpw1, _CH)
        a1 = jnp.pad(a1, ((0, 0), (0, cpw_max - cpw1), (0, 0)))
        return jnp.stack([a0, a1])

    src_r = _core_split(edge_index[0], jnp.zeros((e_pad1,), jnp.int32))
    dst_r = _core_split(edge_index[1], pad_dst)

    rpt = n_pad // 16
    zrows = jnp.zeros((rpt, d), jnp.float32)
    ones_n = jnp.ones((n, d), jnp.float32)

    degp = _seg_sum(ones_n, src_r, dst_r, zrows, n_pad, cpw0, cpw1)
    degp = degp.reshape(2, n_pad, d)[:, :, :16]

    h, g = _tc_pre(x, W_pre, b_pre.reshape(1, d), degp, n_pad)

    for wk, bb, gam, bet, last in ((W_mp1, b_mp1, g1, bt1, False),
                                   (W_mp2, b_mp2, g2, bt2, True)):
        p1 = _seg_sum(g, src_r, dst_r, zrows, n_pad, cpw0, cpw1)
        g = _tc_combine(p1, degp, n, n_pad)
        p2 = _seg_sum(g, src_r, dst_r, zrows, n_pad, cpw0, cpw1)
        g = _tc_combine(p2, degp, n, n_pad)
        p3 = _seg_sum(g, src_r, dst_r, zrows, n_pad, cpw0, cpw1)
        if last:
            return _tc_tag_final_post(h, p1, p2, p3, degp, wk, bb, gam, bet,
                                      W_po1, b_po1, W_po2, b_po2, n_pad)
        h, g = _tc_tag_final(h, p1, p2, p3, degp, wk, bb, gam, bet, n_pad)


# speedup vs baseline: 1.2337x; 1.1529x over previous
import jax
import jax.numpy as jnp
from jax import lax
from jax.experimental import pallas as pl
from jax.experimental.pallas import tpu as pltpu
from jax.experimental.pallas import tpu_sc as plsc

_NW = 32
_CH = 128


def _seg_sum(g, src_r, dst_r, zrows, n_pad, cpw0, cpw1):
    n, d = g.shape
    rpt = n_pad // 16

    def body(g_hbm, src_hbm, dst_hbm, z_hbm, out_hbm, src_v, dst_v, rows_v,
             acc, sem):
        c = lax.axis_index("c")
        s = lax.axis_index("s")
        pltpu.sync_copy(src_hbm.at[c, s], src_v)
        pltpu.sync_copy(dst_hbm.at[c, s], dst_v)
        pltpu.sync_copy(z_hbm, acc.at[pl.ds(s * rpt, rpt)])
        plsc.subcore_barrier()

        def step(j, carry):
            pltpu.async_copy(g_hbm.at[src_v.at[j]], rows_v, sem).wait()
            pltpu.sync_copy(rows_v, acc.at[dst_v.at[j]], add=True)
            return carry

        lax.fori_loop(0, jnp.where(c == 0, cpw0, cpw1), step, 0)
        plsc.subcore_barrier()
        pltpu.sync_copy(acc.at[pl.ds(s * rpt, rpt)],
                        out_hbm.at[pl.ds(c * n_pad + s * rpt, rpt)])

    return pl.kernel(
        body,
        out_type=jax.ShapeDtypeStruct((2 * n_pad, d), jnp.float32),
        mesh=plsc.VectorSubcoreMesh(core_axis_name="c", subcore_axis_name="s"),
        scratch_types=[
            pltpu.VMEM((src_r.shape[2], _CH), jnp.int32),
            pltpu.VMEM((src_r.shape[2], _CH), jnp.int32),
            pltpu.VMEM((_CH, d), jnp.float32),
            pltpu.VMEM_SHARED((n_pad, d), jnp.float32),
            pltpu.SemaphoreType.DMA,
        ],
    )(g, src_r, dst_r, zrows)


_R = 1000


def _dis_block(degp_ref):
    deg = degp_ref[0, :, :1] + degp_ref[1, :, :1]
    return jnp.where(deg > 0, lax.rsqrt(jnp.maximum(deg, 1.0)), 0.0)


def _tc_pre(x, w, b, degp, n_pad):
    n, d = x.shape

    def body(x_ref, w_ref, b_ref, degp_ref, h_ref, g_ref):
        h = jnp.dot(x_ref[...], w_ref[...], preferred_element_type=jnp.float32)
        h = jnp.maximum(h + b_ref[...], 0.0)
        dis = _dis_block(degp_ref)
        h_ref[...] = h
        g_ref[...] = h * dis

    return pl.pallas_call(
        body,
        grid=(n // _R,),
        in_specs=[
            pl.BlockSpec((_R, d), lambda i: (i, 0)),
            pl.BlockSpec((d, d), lambda i: (0, 0)),
            pl.BlockSpec((1, d), lambda i: (0, 0)),
            pl.BlockSpec((2, _R, 16), lambda i: (0, i, 0)),
        ],
        out_specs=[pl.BlockSpec((_R, d), lambda i: (i, 0))] * 2,
        out_shape=[jax.ShapeDtypeStruct((n, d), jnp.float32)] * 2,
    )(x, w, b, degp.reshape(2, n_pad, -1))


def _tc_combine(p, degp, n, n_pad):
    d = p.shape[-1]

    def body(p_ref, degp_ref, g_ref):
        dis = _dis_block(degp_ref)
        g_ref[...] = (p_ref[0] + p_ref[1]) * (dis * dis)

    return pl.pallas_call(
        body,
        grid=(n // _R,),
        in_specs=[
            pl.BlockSpec((2, _R, d), lambda i: (0, i, 0)),
            pl.BlockSpec((2, _R, 16), lambda i: (0, i, 0)),
        ],
        out_specs=pl.BlockSpec((_R, d), lambda i: (i, 0)),
        out_shape=jax.ShapeDtypeStruct((n, d), jnp.float32),
    )(p.reshape(2, n_pad, d), degp.reshape(2, n_pad, -1))


def _tag_block(h_ref, p_refs, degp_ref, wk_ref, b_ref, gam_ref, bet_ref):
    dis = _dis_block(degp_ref)
    acc = jnp.dot(h_ref[...], wk_ref[0], preferred_element_type=jnp.float32)
    for k, p_ref in enumerate(p_refs, start=1):
        hk = (p_ref[0] + p_ref[1]) * dis
        acc = acc + jnp.dot(hk, wk_ref[k], preferred_element_type=jnp.float32)
    acc = jnp.maximum(acc + b_ref[...], 0.0)
    mu = jnp.mean(acc, axis=-1, keepdims=True)
    var = jnp.mean((acc - mu) ** 2, axis=-1, keepdims=True)
    y = (acc - mu) / jnp.sqrt(var + 1e-5) * gam_ref[...] + bet_ref[...]
    return y, dis


def _tc_tag_final(h, p1, p2, p3, degp, wk, b, gam, bet, n_pad):
    n, d = h.shape
    kk = wk.shape[0]

    def body(h_ref, p1_ref, p2_ref, p3_ref, degp_ref, wk_ref, b_ref, gam_ref,
             bet_ref, hn_ref, gn_ref):
        y, dis = _tag_block(h_ref, (p1_ref, p2_ref, p3_ref), degp_ref, wk_ref,
                            b_ref, gam_ref, bet_ref)
        hn_ref[...] = y
        gn_ref[...] = y * dis

    return pl.pallas_call(
        body,
        grid=(n // _R,),
        in_specs=[
            pl.BlockSpec((_R, d), lambda i: (i, 0)),
            pl.BlockSpec((2, _R, d), lambda i: (0, i, 0)),
            pl.BlockSpec((2, _R, d), lambda i: (0, i, 0)),
            pl.BlockSpec((2, _R, d), lambda i: (0, i, 0)),
            pl.BlockSpec((2, _R, 16), lambda i: (0, i, 0)),
            pl.BlockSpec((kk, d, d), lambda i: (0, 0, 0)),
            pl.BlockSpec((1, d), lambda i: (0, 0)),
            pl.BlockSpec((1, d), lambda i: (0, 0)),
            pl.BlockSpec((1, d), lambda i: (0, 0)),
        ],
        out_specs=[pl.BlockSpec((_R, d), lambda i: (i, 0))] * 2,
        out_shape=[jax.ShapeDtypeStruct((n, d), jnp.float32)] * 2,
    )(h, p1.reshape(2, n_pad, d), p2.reshape(2, n_pad, d),
      p3.reshape(2, n_pad, d), degp.reshape(2, n_pad, -1), wk,
      b.reshape(1, d), gam.reshape(1, d), bet.reshape(1, d))


def _tc_tag_final_post(h, p1, p2, p3, degp, wk, b, gam, bet,
                       w_po1, b_po1, w_po2, b_po2, n_pad):
    n, d = h.shape
    kk = wk.shape[0]

    def body(h_ref, p1_ref, p2_ref, p3_ref, degp_ref, wk_ref, b_ref, gam_ref,
             bet_ref, w1_ref, b1_ref, w2_ref, b2_ref, out_ref):
        y, _ = _tag_block(h_ref, (p1_ref, p2_ref, p3_ref), degp_ref, wk_ref,
                          b_ref, gam_ref, bet_ref)
        hp = jnp.dot(y, w1_ref[...], preferred_element_type=jnp.float32)
        hp = jnp.maximum(hp + b1_ref[...], 0.0)
        out_ref[...] = (jnp.dot(hp, w2_ref[...],
                                preferred_element_type=jnp.float32)
                        + b2_ref[...])

    return pl.pallas_call(
        body,
        grid=(n // _R,),
        in_specs=[
            pl.BlockSpec((_R, d), lambda i: (i, 0)),
            pl.BlockSpec((2, _R, d), lambda i: (0, i, 0)),
            pl.BlockSpec((2, _R, d), lambda i: (0, i, 0)),
            pl.BlockSpec((2, _R, d), lambda i: (0, i, 0)),
            pl.BlockSpec((2, _R, 16), lambda i: (0, i, 0)),
            pl.BlockSpec((kk, d, d), lambda i: (0, 0, 0)),
            pl.BlockSpec((1, d), lambda i: (0, 0)),
            pl.BlockSpec((1, d), lambda i: (0, 0)),
            pl.BlockSpec((1, d), lambda i: (0, 0)),
            pl.BlockSpec((d, d), lambda i: (0, 0)),
            pl.BlockSpec((1, d), lambda i: (0, 0)),
            pl.BlockSpec((d, 1), lambda i: (0, 0)),
            pl.BlockSpec((1, 1), lambda i: (0, 0)),
        ],
        out_specs=pl.BlockSpec((_R, 1), lambda i: (i, 0)),
        out_shape=jax.ShapeDtypeStruct((n, 1), jnp.float32),
    )(h, p1.reshape(2, n_pad, d), p2.reshape(2, n_pad, d),
      p3.reshape(2, n_pad, d), degp.reshape(2, n_pad, -1), wk,
      b.reshape(1, d), gam.reshape(1, d), bet.reshape(1, d),
      w_po1, b_po1.reshape(1, d), w_po2, b_po2.reshape(1, 1))


def kernel(x, edge_index, W_pre, b_pre, W_mp1, b_mp1, g1, bt1,
           W_mp2, b_mp2, g2, bt2, W_po1, b_po1, W_po2, b_po2):
    n, d = x.shape
    e = edge_index.shape[1]

    n_pad = -(-(n + 1) // 128) * 128

    ept = 16 * _CH
    cpw0 = max(1, round(e * 0.41 / ept))
    e0 = ept * cpw0
    cpw1 = -(-(e - e0) // ept)
    cpw_max = max(cpw0, cpw1)

    e_pad1 = ept * cpw1 - (e - e0)
    pad_dst = n + (jnp.arange(e_pad1, dtype=jnp.int32) % (n_pad - n))

    def _core_split(idx, pad_vals):
        a0 = idx[:e0].reshape(16, cpw0, _CH)
        a0 = jnp.pad(a0, ((0, 0), (0, cpw_max - cpw0), (0, 0)))
        a1 = jnp.concatenate([idx[e0:], pad_vals]).reshape(16, cpw1, _CH)
        a1 = jnp.pad(a1, ((0, 0), (0, cpw_max - cpw1), (0, 0)))
        return jnp.stack([a0, a1])

    src_r = _core_split(edge_index[0], jnp.zeros((e_pad1,), jnp.int32))
    dst_r = _core_split(edge_index[1], pad_dst)

    rpt = n_pad // 16
    zrows = jnp.zeros((rpt, d), jnp.float32)
    ones_n = jnp.ones((n, d), jnp.float32)

    degp = _seg_sum(ones_n, src_r, dst_r, zrows, n_pad, cpw0, cpw1)
    degp = degp.reshape(2, n_pad, d)[:, :, :16]

    h, g = _tc_pre(x, W_pre, b_pre.reshape(1, d), degp, n_pad)

    for wk, bb, gam, bet, last in ((W_mp1, b_mp1, g1, bt1, False),
                                   (W_mp2, b_mp2, g2, bt2, True)):
        p1 = _seg_sum(g, src_r, dst_r, zrows, n_pad, cpw0, cpw1)
        g = _tc_combine(p1, degp, n, n_pad)
        p2 = _seg_sum(g, src_r, dst_r, zrows, n_pad, cpw0, cpw1)
        g = _tc_combine(p2, degp, n, n_pad)
        p3 = _seg_sum(g, src_r, dst_r, zrows, n_pad, cpw0, cpw1)
        if last:
            return _tc_tag_final_post(h, p1, p2, p3, degp, wk, bb, gam, bet,
                                      W_po1, b_po1, W_po2, b_po2, n_pad)
        h, g = _tc_tag_final(h, p1, p2, p3, degp, wk, bb, gam, bet, n_pad)
